# baseline (speedup 1.0000x reference)
import functools

import jax
import jax.numpy as jnp
from jax import lax
from jax.experimental import pallas as pl
from jax.experimental.pallas import tpu as pltpu
from jax.experimental.pallas import tpu_sc as plsc

_NC = 2
_NS = 16
_NBUF = 4
_LOOK = 2
_C = 16


def kernel(tokens, token_embedding, position_embedding):
    Bt, T = tokens.shape
    V, D = token_embedding.shape
    NW = _NC * _NS
    b_per_w = Bt // NW
    n_p = b_per_w // _C
    n_tasks = T * n_p
    NVEC = D // 16

    idx = (tokens.astype(jnp.int32).T
           .reshape(T, NW, b_per_w).transpose(1, 0, 2))

    mesh = plsc.VectorSubcoreMesh(core_axis_name="c", subcore_axis_name="s")

    @functools.partial(
        pl.kernel,
        out_type=jax.ShapeDtypeStruct((T, Bt, D), jnp.float32),
        mesh=mesh,
        scratch_types=[
            pltpu.VMEM((T, b_per_w), jnp.int32),
            pltpu.VMEM((T * D,), jnp.float32),
            [pltpu.VMEM((_C, D), jnp.float32) for _ in range(_NBUF)],
            [pltpu.SemaphoreType.DMA for _ in range(_NBUF)],
            [pltpu.SemaphoreType.DMA for _ in range(_NBUF)],
        ],
    )
    def body(idx_hbm, table_hbm, pos_hbm, out_hbm,
             idx_v, pos_v, bufs, gsems, wsems):
        wid = lax.axis_index("s") * _NC + lax.axis_index("c")
        pltpu.sync_copy(idx_hbm.at[wid], idx_v)
        pltpu.sync_copy(pos_hbm, pos_v)
        b0 = wid * b_per_w

        def fire_gather(m, slot):
            t = lax.div(m, n_p)
            p = lax.rem(m, n_p)
            iv = idx_v[t, pl.ds(pl.multiple_of(p * _C, _C), _C)]
            pltpu.async_copy(table_hbm.at[iv], bufs[slot], gsems[slot])

        def wait_gather(m, slot):
            iv = idx_v[0, pl.ds(0, _C)]
            pltpu.make_async_copy(table_hbm.at[iv], bufs[slot],
                                  gsems[slot]).wait()

        def fire_wb(m, slot):
            t = lax.div(m, n_p)
            p = lax.rem(m, n_p)
            pltpu.async_copy(bufs[slot],
                             out_hbm.at[t, pl.ds(b0 + p * _C, _C)],
                             wsems[slot])

        def wait_wb(slot):
            pltpu.make_async_copy(bufs[slot], out_hbm.at[0, pl.ds(b0, _C)],
                                  wsems[slot]).wait()

        def add_pos(m, slot):
            pbase = pl.multiple_of(lax.div(m, n_p) * D, 16)

            def col_fn(j, carry):
                jc = pl.multiple_of(j * 16, 16)
                pv = pos_v[pl.ds(pbase + jc, 16)]
                for r in range(_C):
                    plsc.addupdate(bufs[slot].at[r, pl.ds(jc, 16)], pv)
                return carry

            lax.fori_loop(0, NVEC, col_fn, 0)

        def step(m, s, *, wait_w, fire_g):
            t = (s + _LOOK) % _NBUF
            if fire_g:
                if wait_w:
                    wait_wb(t)
                fire_gather(m + _LOOK, t)
            wait_gather(m, s)
            add_pos(m, s)
            fire_wb(m, s)

        for m0 in range(_LOOK):
            fire_gather(m0, m0)

        for s in range(_NBUF):
            step(s, s, wait_w=(s + _LOOK >= _NBUF), fire_g=True)

        n_rounds = n_tasks // _NBUF

        def round_body(i, carry):
            for s in range(_NBUF):
                step(i * _NBUF + s, s, wait_w=True, fire_g=True)
            return carry

        lax.fori_loop(1, n_rounds, round_body, 0)

        for m in range(n_rounds * _NBUF, n_tasks):
            s = m % _NBUF
            step(m, s, wait_w=(m + _LOOK < n_tasks),
                 fire_g=(m + _LOOK < n_tasks))

        for s in range(_NBUF):
            wait_wb(s)

    out = body(idx, token_embedding, position_embedding.reshape(-1))
    return jnp.transpose(out, (1, 0, 2))
